# Initial kernel scaffold; baseline (speedup 1.0000x reference)
import jax
import jax.numpy as jnp
from jax.experimental import pallas as pl


def kernel(x, t, Wt, bt, W0, b0, W1, b1, W2, b2):
    raise NotImplementedError("write your pallas kernel here")


# speedup vs baseline: 8.2330x; 8.2330x over previous
import math

import jax
import jax.numpy as jnp
from jax.experimental import pallas as pl
from jax.experimental.pallas import tpu as pltpu


def _trunk_body(coords_ref, t_ref, Wt_ref, bt_ref, w0c_ref, w0pe_ref,
                w0t_ref, b0_ref, W1_ref, b1_ref, W2_ref, b2_ref, out_ref):
    f32 = jnp.float32
    gb = t_ref.shape[0]
    nv = coords_ref.shape[0] // gb
    hid = out_ref.shape[1]
    tdim = Wt_ref.shape[0]
    third = f32(1.0 / 3.0)

    half = tdim // 2
    freqs = jnp.exp(
        jax.lax.broadcasted_iota(jnp.int32, (1, half), 1).astype(f32)
        * f32(-math.log(10000.0) / (half - 1)))
    emb = t_ref[...] * freqs
    sc = jnp.concatenate([jnp.sin(emb), jnp.cos(emb)], axis=-1)
    temb = jax.nn.silu(
        jnp.dot(sc, Wt_ref[...], preferred_element_type=f32) + bt_ref[0, :])
    tw = jnp.dot(temb, w0t_ref[...], preferred_element_type=f32) + b0_ref[0, :]

    v = jax.lax.broadcasted_iota(jnp.int32, (nv, 1), 0).astype(f32)
    a1 = f32(2.0 * math.pi / nv) * v
    pew = (jnp.sin(a1) * w0pe_ref[0, :][None, :]
           + jnp.cos(a1) * w0pe_ref[1, :][None, :]
           + jnp.sin(2.0 * a1) * w0pe_ref[2, :][None, :]
           + jnp.cos(2.0 * a1) * w0pe_ref[3, :][None, :])

    c = jnp.dot(coords_ref[...], w0c_ref[...], preferred_element_type=f32)
    s0 = c.reshape(gb, nv, hid) + pew[None, :, :] + tw[:, None, :]

    def cycle_agg_silu(s3):
        up = jnp.concatenate([s3[:, -1:, :], s3[:, :-1, :]], axis=1)
        dn = jnp.concatenate([s3[:, 1:, :], s3[:, :1, :]], axis=1)
        return jax.nn.silu((s3 + up + dn) * third)

    h = cycle_agg_silu(s0)

    def gcn_layer(h3, W_ref, b_ref):
        s = jnp.dot(h3.reshape(gb * nv, hid), W_ref[...],
                    preferred_element_type=f32) + b_ref[0, :]
        return cycle_agg_silu(s.reshape(gb, nv, hid))

    h = gcn_layer(h, W1_ref, b1_ref)
    h = gcn_layer(h, W2_ref, b2_ref)

    out_ref[...] = jnp.sum(h, axis=1) * f32(1.0 / nv)


def kernel(x, t, Wt, bt, W0, b0, W1, b1, W2, b2):
    bsz, d = x.shape
    nv = d // 2
    hid = W1.shape[0]
    tdim = Wt.shape[0]
    GB = 16
    grid = bsz // GB

    coords = x.reshape(bsz * nv, 2)
    t2 = t.reshape(bsz, 1)
    w0c, w0pe, w0t = W0[:2], W0[2:6], W0[6:]
    b0r, b1r, b2r, btr = (b0.reshape(1, -1), b1.reshape(1, -1),
                          b2.reshape(1, -1), bt.reshape(1, -1))

    const = lambda *shape: pl.BlockSpec(shape, lambda i: (0,) * len(shape))
    out = pl.pallas_call(
        _trunk_body,
        grid=(grid,),
        in_specs=[
            pl.BlockSpec((GB * nv, 2), lambda i: (i, 0)),
            pl.BlockSpec((GB, 1), lambda i: (i, 0)),
            const(tdim, tdim),
            const(1, tdim),
            const(2, hid),
            const(4, hid),
            const(tdim, hid),
            const(1, hid),
            const(hid, hid),
            const(1, hid),
            const(hid, hid),
            const(1, hid),
        ],
        out_specs=pl.BlockSpec((GB, hid), lambda i: (i, 0)),
        out_shape=jax.ShapeDtypeStruct((bsz, hid), jnp.float32),
        compiler_params=pltpu.CompilerParams(
            dimension_semantics=("arbitrary",)),
    )(coords, t2, Wt, btr, w0c, w0pe, w0t, b0r, W1, b1r, W2, b2r)
    return out


# speedup vs baseline: 9.7760x; 1.1874x over previous
import math

import jax
import jax.numpy as jnp
from jax.experimental import pallas as pl
from jax.experimental.pallas import tpu as pltpu


def _trunk_body(coords_ref, t_ref, Wt_ref, bt_ref, w0c_ref, w0pe_ref,
                w0t_ref, b0_ref, W1_ref, b1_ref, W2_ref, b2_ref, out_ref):
    f32 = jnp.float32
    gb = t_ref.shape[0]
    nv = coords_ref.shape[0] // gb
    hid = out_ref.shape[1]
    tdim = Wt_ref.shape[0]

    vi = jax.lax.broadcasted_iota(jnp.int32, (nv, nv), 0)
    vj = jax.lax.broadcasted_iota(jnp.int32, (nv, nv), 1)
    dd = jnp.abs(vi - vj)
    T = jnp.where((dd <= 1) | (dd == nv - 1), f32(1.0), f32(0.0))

    half = tdim // 2
    freqs = jnp.exp(
        jax.lax.broadcasted_iota(jnp.int32, (1, half), 1).astype(f32)
        * f32(-math.log(10000.0) / (half - 1)))
    emb = t_ref[...] * freqs
    sc = jnp.concatenate([jnp.sin(emb), jnp.cos(emb)], axis=-1)
    temb = jax.nn.silu(
        jnp.dot(sc, Wt_ref[...], preferred_element_type=f32) + bt_ref[0, :])
    tw = jnp.dot(temb, w0t_ref[...], preferred_element_type=f32) + b0_ref[0, :]

    v = jax.lax.broadcasted_iota(jnp.int32, (nv, 1), 0).astype(f32)
    a1 = f32(2.0 * math.pi / nv) * v
    sc1 = f32((1.0 + 2.0 * math.cos(2.0 * math.pi / nv)) / 3.0)
    sc2 = f32((1.0 + 2.0 * math.cos(4.0 * math.pi / nv)) / 3.0)
    pew = (jnp.sin(a1) * sc1 * w0pe_ref[0, :][None, :]
           + jnp.cos(a1) * sc1 * w0pe_ref[1, :][None, :]
           + jnp.sin(2.0 * a1) * sc2 * w0pe_ref[2, :][None, :]
           + jnp.cos(2.0 * a1) * sc2 * w0pe_ref[3, :][None, :])

    def cycle_mix(s3):
        return jnp.stack(
            [jnp.dot(T, s3[g], preferred_element_type=f32)
             for g in range(gb)], axis=0)

    c = jnp.dot(coords_ref[...], w0c_ref[...], preferred_element_type=f32)
    h = jax.nn.silu(cycle_mix(c.reshape(gb, nv, hid))
                    + pew[None, :, :] + tw[:, None, :])

    def gcn_layer(h3, W_ref, b_ref):
        s = jnp.dot(h3.reshape(gb * nv, hid), W_ref[...],
                    preferred_element_type=f32)
        return jax.nn.silu(cycle_mix(s.reshape(gb, nv, hid)) + b_ref[0, :])

    h = gcn_layer(h, W1_ref, b1_ref)
    h = gcn_layer(h, W2_ref, b2_ref)

    out_ref[...] = jnp.sum(h, axis=1) * f32(1.0 / nv)


def kernel(x, t, Wt, bt, W0, b0, W1, b1, W2, b2):
    bsz, d = x.shape
    nv = d // 2
    hid = W1.shape[0]
    tdim = Wt.shape[0]
    GB = 16
    grid = bsz // GB

    coords = x.reshape(bsz * nv, 2)
    t2 = t.reshape(bsz, 1)
    third = 1.0 / 3.0
    w0c, w0pe, w0t = W0[:2] * third, W0[2:6], W0[6:]
    W1s, W2s = W1 * third, W2 * third
    b0r, b1r, b2r, btr = (b0.reshape(1, -1), b1.reshape(1, -1),
                          b2.reshape(1, -1), bt.reshape(1, -1))

    const = lambda *shape: pl.BlockSpec(shape, lambda i: (0,) * len(shape))
    out = pl.pallas_call(
        _trunk_body,
        grid=(grid,),
        in_specs=[
            pl.BlockSpec((GB * nv, 2), lambda i: (i, 0)),
            pl.BlockSpec((GB, 1), lambda i: (i, 0)),
            const(tdim, tdim),
            const(1, tdim),
            const(2, hid),
            const(4, hid),
            const(tdim, hid),
            const(1, hid),
            const(hid, hid),
            const(1, hid),
            const(hid, hid),
            const(1, hid),
        ],
        out_specs=pl.BlockSpec((GB, hid), lambda i: (i, 0)),
        out_shape=jax.ShapeDtypeStruct((bsz, hid), jnp.float32),
        compiler_params=pltpu.CompilerParams(
            dimension_semantics=("arbitrary",)),
    )(coords, t2, Wt, btr, w0c, w0pe, w0t, b0r, W1s, b1r, W2s, b2r)
    return out


# speedup vs baseline: 13.4381x; 1.3746x over previous
import math

import jax
import jax.numpy as jnp
import numpy as np
from jax.experimental import pallas as pl
from jax.experimental.pallas import tpu as pltpu


def _silu(x):
    return (0.5 * x) * (1.0 + jnp.tanh(0.5 * x))


def _trunk_body(coords_ref, t_ref, freqs_ref, T_ref, pew_ref, Wt_ref, bt_ref,
                w0c_ref, w0t_ref, b0_ref, W1_ref, b1_ref, W2_ref, b2_ref,
                out_ref):
    f32 = jnp.float32
    gb = t_ref.shape[0]
    nv = coords_ref.shape[0] // gb
    hid = out_ref.shape[1]

    emb = t_ref[...] * freqs_ref[0, :]
    sc = jnp.concatenate([jnp.sin(emb), jnp.cos(emb)], axis=-1)
    temb = _silu(
        jnp.dot(sc, Wt_ref[...], preferred_element_type=f32) + bt_ref[0, :])
    tw = jnp.dot(temb, w0t_ref[...], preferred_element_type=f32) + b0_ref[0, :]

    T = T_ref[...]

    def cycle_mix(s3):
        return jnp.stack(
            [jnp.dot(T, s3[g], preferred_element_type=f32)
             for g in range(gb)], axis=0)

    c = jnp.dot(coords_ref[...], w0c_ref[...], preferred_element_type=f32)
    h = _silu(cycle_mix(c.reshape(gb, nv, hid))
              + pew_ref[...][None, :, :] + tw[:, None, :])

    def gcn_layer(h3, W_ref, b_ref):
        s = jnp.dot(h3.reshape(gb * nv, hid), W_ref[...],
                    preferred_element_type=f32)
        return _silu(cycle_mix(s.reshape(gb, nv, hid)) + b_ref[0, :])

    h = gcn_layer(h, W1_ref, b1_ref)
    h = gcn_layer(h, W2_ref, b2_ref)

    out_ref[...] = jnp.sum(h, axis=1) * f32(1.0 / nv)


def kernel(x, t, Wt, bt, W0, b0, W1, b1, W2, b2):
    bsz, d = x.shape
    nv = d // 2
    hid = W1.shape[0]
    tdim = Wt.shape[0]
    GB = 32
    grid = bsz // GB
    f32 = jnp.float32

    coords = x.reshape(bsz * nv, 2)
    t2 = t.reshape(bsz, 1)
    third = 1.0 / 3.0

    half = tdim // 2
    freqs = jnp.exp(jnp.arange(half, dtype=f32)
                    * (-math.log(10000.0) / (half - 1))).reshape(1, half)
    vidx = jnp.arange(nv)
    dd = jnp.abs(vidx[:, None] - vidx[None, :])
    T = ((dd <= 1) | (dd == nv - 1)).astype(f32)
    frac = vidx.astype(f32) / nv
    pe = jnp.stack([jnp.sin(2.0 * np.pi * frac), jnp.cos(2.0 * np.pi * frac),
                    jnp.sin(4.0 * np.pi * frac), jnp.cos(4.0 * np.pi * frac)],
                   axis=-1)
    sc1 = (1.0 + 2.0 * math.cos(2.0 * math.pi / nv)) / 3.0
    sc2 = (1.0 + 2.0 * math.cos(4.0 * math.pi / nv)) / 3.0
    scale = jnp.array([sc1, sc1, sc2, sc2], dtype=f32)
    pew = (pe * scale[None, :]) @ W0[2:6]

    w0c, w0t = W0[:2] * third, W0[6:]
    W1s, W2s = W1 * third, W2 * third
    b0r, b1r, b2r, btr = (b0.reshape(1, -1), b1.reshape(1, -1),
                          b2.reshape(1, -1), bt.reshape(1, -1))

    const = lambda *shape: pl.BlockSpec(shape, lambda i: (0,) * len(shape))
    out = pl.pallas_call(
        _trunk_body,
        grid=(grid,),
        in_specs=[
            pl.BlockSpec((GB * nv, 2), lambda i: (i, 0)),
            pl.BlockSpec((GB, 1), lambda i: (i, 0)),
            const(1, half),
            const(nv, nv),
            const(nv, hid),
            const(tdim, tdim),
            const(1, tdim),
            const(2, hid),
            const(tdim, hid),
            const(1, hid),
            const(hid, hid),
            const(1, hid),
            const(hid, hid),
            const(1, hid),
        ],
        out_specs=pl.BlockSpec((GB, hid), lambda i: (i, 0)),
        out_shape=jax.ShapeDtypeStruct((bsz, hid), jnp.float32),
        compiler_params=pltpu.CompilerParams(
            dimension_semantics=("arbitrary",)),
    )(coords, t2, freqs, T, pew, Wt, btr, w0c, w0t, b0r, W1s, b1r, W2s, b2r)
    return out
